# baseline (speedup 1.0000x reference)
import functools

import jax
import jax.numpy as jnp
from jax import lax
from jax.experimental import pallas as pl
from jax.experimental.pallas import tpu as pltpu
from jax.experimental.pallas import tpu_sc as plsc

NUM_NODES = 16384
DIM = 128
CHUNK = 128
NBUF = 3


@functools.lru_cache(maxsize=None)
def _build():
    info = plsc.get_sparse_core_info()
    nc, ns = info.num_cores, info.num_subcores
    nw = nc * ns
    b_per_w = NUM_NODES // nw
    nchunk = b_per_w // CHUNK
    mesh = plsc.VectorSubcoreMesh(core_axis_name="c", subcore_axis_name="s")

    @functools.partial(
        pl.kernel,
        mesh=mesh,
        out_type=jax.ShapeDtypeStruct((NUM_NODES, DIM), jnp.float32),
        scratch_types=[
            pltpu.VMEM((b_per_w,), jnp.int32),
            pltpu.VMEM_SHARED((9, DIM), jnp.float32),
        ]
        + [pltpu.VMEM((CHUNK, DIM), jnp.float32)] * NBUF
        + [pltpu.SemaphoreType.DMA] * (2 * NBUF + 1),
    )
    def gather_kernel(idx_hbm, table_hbm, out_hbm, idx_v, table_sh, *rest):
        bufs = rest[:NBUF]
        gsems = rest[NBUF:2 * NBUF]
        ssems = rest[2 * NBUF:3 * NBUF]
        isem = rest[3 * NBUF]

        sid = lax.axis_index("s")
        wid = sid * nc + lax.axis_index("c")
        base = wid * b_per_w

        hidx = pltpu.async_copy(idx_hbm.at[pl.ds(base, b_per_w)], idx_v, isem)

        @pl.when(sid == 0)
        def _stage_table():
            pltpu.sync_copy(table_hbm, table_sh)

        hidx.wait()
        plsc.subcore_barrier()

        def gather(k):
            return pltpu.async_copy(
                table_sh.at[idx_v.at[pl.ds(k * CHUNK, CHUNK)]],
                bufs[k % NBUF], gsems[k % NBUF])

        def scatter(k):
            return pltpu.async_copy(
                bufs[k % NBUF], out_hbm.at[pl.ds(base + k * CHUNK, CHUNK)],
                ssems[k % NBUF])

        hg = [None] * nchunk
        hs = [None] * nchunk
        for k in range(min(NBUF, nchunk)):
            hg[k] = gather(k)
        for k in range(nchunk):
            hg[k].wait()
            hs[k] = scatter(k)
            nxt = k + NBUF
            if nxt < nchunk:
                hs[nxt - NBUF].wait()
                hg[nxt] = gather(nxt)
        for k in range(max(0, nchunk - NBUF), nchunk):
            hs[k].wait()

    return gather_kernel


def kernel(abstract_type_ids, embedding_table):
    ids = abstract_type_ids.astype(jnp.int32)
    return _build()(ids, embedding_table)


# speedup vs baseline: 1.0062x; 1.0062x over previous
import functools

import jax
import jax.numpy as jnp
from jax import lax
from jax.experimental import pallas as pl
from jax.experimental.pallas import tpu as pltpu
from jax.experimental.pallas import tpu_sc as plsc

NUM_NODES = 16384
DIM = 128
CHUNK = 64
NBUF = 4


@functools.lru_cache(maxsize=None)
def _build():
    info = plsc.get_sparse_core_info()
    nc, ns = info.num_cores, info.num_subcores
    nw = nc * ns
    b_per_w = NUM_NODES // nw
    nchunk = b_per_w // CHUNK
    mesh = plsc.VectorSubcoreMesh(core_axis_name="c", subcore_axis_name="s")

    @functools.partial(
        pl.kernel,
        mesh=mesh,
        out_type=jax.ShapeDtypeStruct((NUM_NODES, DIM), jnp.float32),
        scratch_types=[
            pltpu.VMEM((b_per_w,), jnp.int32),
            pltpu.VMEM_SHARED((9, DIM), jnp.float32),
        ]
        + [pltpu.VMEM((CHUNK, DIM), jnp.float32)] * NBUF
        + [pltpu.SemaphoreType.DMA] * (2 * NBUF + 1),
    )
    def gather_kernel(idx_hbm, table_hbm, out_hbm, idx_v, table_sh, *rest):
        bufs = rest[:NBUF]
        gsems = rest[NBUF:2 * NBUF]
        ssems = rest[2 * NBUF:3 * NBUF]
        isem = rest[3 * NBUF]

        sid = lax.axis_index("s")
        wid = sid * nc + lax.axis_index("c")
        base = wid * b_per_w

        hidx = pltpu.async_copy(idx_hbm.at[pl.ds(base, b_per_w)], idx_v, isem)

        @pl.when(sid == 0)
        def _stage_table():
            pltpu.sync_copy(table_hbm, table_sh)

        hidx.wait()
        plsc.subcore_barrier()

        def gather(k):
            return pltpu.async_copy(
                table_sh.at[idx_v.at[pl.ds(k * CHUNK, CHUNK)]],
                bufs[k % NBUF], gsems[k % NBUF])

        def scatter(k):
            return pltpu.async_copy(
                bufs[k % NBUF], out_hbm.at[pl.ds(base + k * CHUNK, CHUNK)],
                ssems[k % NBUF])

        hg = [None] * nchunk
        hs = [None] * nchunk
        for k in range(min(NBUF, nchunk)):
            hg[k] = gather(k)
        for k in range(nchunk):
            hg[k].wait()
            hs[k] = scatter(k)
            nxt = k + NBUF
            if nxt < nchunk:
                hs[nxt - NBUF].wait()
                hg[nxt] = gather(nxt)
        for k in range(max(0, nchunk - NBUF), nchunk):
            hs[k].wait()

    return gather_kernel


def kernel(abstract_type_ids, embedding_table):
    ids = abstract_type_ids.astype(jnp.int32)
    return _build()(ids, embedding_table)


# speedup vs baseline: 1.0073x; 1.0011x over previous
import functools

import jax
import jax.numpy as jnp
from jax import lax
from jax.experimental import pallas as pl
from jax.experimental.pallas import tpu as pltpu
from jax.experimental.pallas import tpu_sc as plsc

NUM_NODES = 16384
DIM = 128
CHUNK = 64
NBUF = 3


@functools.lru_cache(maxsize=None)
def _build():
    info = plsc.get_sparse_core_info()
    nc, ns = info.num_cores, info.num_subcores
    nw = nc * ns
    b_per_w = NUM_NODES // nw
    nchunk = b_per_w // CHUNK
    mesh = plsc.VectorSubcoreMesh(core_axis_name="c", subcore_axis_name="s")

    @functools.partial(
        pl.kernel,
        mesh=mesh,
        out_type=jax.ShapeDtypeStruct((NUM_NODES, DIM), jnp.float32),
        scratch_types=[
            pltpu.VMEM((b_per_w,), jnp.int32),
            pltpu.VMEM_SHARED((9, DIM), jnp.float32),
        ]
        + [pltpu.VMEM((CHUNK, DIM), jnp.float32)] * NBUF
        + [pltpu.SemaphoreType.DMA] * (2 * NBUF + 1),
    )
    def gather_kernel(idx_hbm, table_hbm, out_hbm, idx_v, table_sh, *rest):
        bufs = rest[:NBUF]
        gsems = rest[NBUF:2 * NBUF]
        ssems = rest[2 * NBUF:3 * NBUF]
        isem = rest[3 * NBUF]

        sid = lax.axis_index("s")
        wid = sid * nc + lax.axis_index("c")
        base = wid * b_per_w

        hidx = pltpu.async_copy(idx_hbm.at[pl.ds(base, b_per_w)], idx_v, isem)

        @pl.when(sid == 0)
        def _stage_table():
            pltpu.sync_copy(table_hbm, table_sh)

        hidx.wait()
        plsc.subcore_barrier()

        def gather(k):
            return pltpu.async_copy(
                table_sh.at[idx_v.at[pl.ds(k * CHUNK, CHUNK)]],
                bufs[k % NBUF], gsems[k % NBUF])

        def scatter(k):
            return pltpu.async_copy(
                bufs[k % NBUF], out_hbm.at[pl.ds(base + k * CHUNK, CHUNK)],
                ssems[k % NBUF])

        hg = [None] * nchunk
        hs = [None] * nchunk
        for k in range(min(NBUF, nchunk)):
            hg[k] = gather(k)
        for k in range(nchunk):
            hg[k].wait()
            hs[k] = scatter(k)
            nxt = k + NBUF
            if nxt < nchunk:
                hs[nxt - NBUF].wait()
                hg[nxt] = gather(nxt)
        for k in range(max(0, nchunk - NBUF), nchunk):
            hs[k].wait()

    return gather_kernel


def kernel(abstract_type_ids, embedding_table):
    ids = abstract_type_ids.astype(jnp.int32)
    return _build()(ids, embedding_table)


# speedup vs baseline: 1.0723x; 1.0646x over previous
import functools

import jax
import jax.numpy as jnp
from jax import lax
from jax.experimental import pallas as pl
from jax.experimental.pallas import tpu as pltpu
from jax.experimental.pallas import tpu_sc as plsc

NUM_NODES = 16384
DIM = 128
CHUNK = 64
NBUF = 3


@functools.lru_cache(maxsize=None)
def _build():
    info = plsc.get_sparse_core_info()
    nc, ns = info.num_cores, info.num_subcores
    nw = nc * ns
    b_per_w = NUM_NODES // nw
    nchunk = b_per_w // CHUNK
    mesh = plsc.VectorSubcoreMesh(core_axis_name="c", subcore_axis_name="s", num_cores=1)

    @functools.partial(
        pl.kernel,
        mesh=mesh,
        out_type=jax.ShapeDtypeStruct((NUM_NODES, DIM), jnp.float32),
        scratch_types=[
            pltpu.VMEM((b_per_w,), jnp.int32),
            pltpu.VMEM_SHARED((9, DIM), jnp.float32),
        ]
        + [pltpu.VMEM((CHUNK, DIM), jnp.float32)] * NBUF
        + [pltpu.SemaphoreType.DMA] * (2 * NBUF + 1),
    )
    def gather_kernel(idx_hbm, table_hbm, out_hbm, idx_v, table_sh, *rest):
        bufs = rest[:NBUF]
        gsems = rest[NBUF:2 * NBUF]
        ssems = rest[2 * NBUF:3 * NBUF]
        isem = rest[3 * NBUF]

        sid = lax.axis_index("s")
        wid = sid * nc + lax.axis_index("c")
        base = wid * b_per_w

        hidx = pltpu.async_copy(idx_hbm.at[pl.ds(base, b_per_w)], idx_v, isem)

        @pl.when(sid == 0)
        def _stage_table():
            pltpu.sync_copy(table_hbm, table_sh)

        hidx.wait()
        plsc.subcore_barrier()

        def gather(k):
            return pltpu.async_copy(
                table_sh.at[idx_v.at[pl.ds(k * CHUNK, CHUNK)]],
                bufs[k % NBUF], gsems[k % NBUF])

        def scatter(k):
            return pltpu.async_copy(
                bufs[k % NBUF], out_hbm.at[pl.ds(base + k * CHUNK, CHUNK)],
                ssems[k % NBUF])

        hg = [None] * nchunk
        hs = [None] * nchunk
        for k in range(min(NBUF, nchunk)):
            hg[k] = gather(k)
        for k in range(nchunk):
            hg[k].wait()
            hs[k] = scatter(k)
            nxt = k + NBUF
            if nxt < nchunk:
                hs[nxt - NBUF].wait()
                hg[nxt] = gather(nxt)
        for k in range(max(0, nchunk - NBUF), nchunk):
            hs[k].wait()

    return gather_kernel


def kernel(abstract_type_ids, embedding_table):
    ids = abstract_type_ids.astype(jnp.int32)
    return _build()(ids, embedding_table)
